# baseline (speedup 1.0000x reference)
import functools

import jax
import jax.numpy as jnp
import numpy as np
from jax import lax
from jax.experimental import pallas as pl
from jax.experimental.pallas import tpu as pltpu
from jax.experimental.pallas import tpu_sc as plsc

B, L, V = 8, 2048, 2048
NEG = float(np.log(np.float32(1e-45)))
R = 1024


def _sc_lastpos(seq):
    mesh = plsc.VectorSubcoreMesh(core_axis_name="c", subcore_axis_name="s", num_cores=1)

    @functools.partial(
        pl.kernel,
        mesh=mesh,
        out_type=jax.ShapeDtypeStruct((2 * B, V), jnp.int32),
        compiler_params=pltpu.CompilerParams(needs_layout_passes=False),
        scratch_types=[
            pltpu.VMEM((L // 2,), jnp.int32),
            pltpu.VMEM((V,), jnp.int32),
            pltpu.VMEM((32,), jnp.int32),
            pltpu.SemaphoreType.DMA,
        ],
    )
    def k(seq_hbm, out_hbm, seq_v, lp_v, tmp_v, dsem):
        wid = lax.axis_index("s")

        @pl.when(wid < 2 * B)
        def _():
            b = wid // 2
            base = (wid % 2) * (L // 2)
            cp = pltpu.make_async_copy(
                seq_hbm.at[b, pl.ds(base, L // 2)], seq_v, dsem)
            cp.start()
            tmp_v[pl.ds(16, 16)] = jnp.full((16,), 0x7FFFFFFF, jnp.int32)

            def init_body(i, _):
                lp_v[pl.ds(i * 16, 16)] = jnp.full((16,), -1, jnp.int32)
                return 0

            lax.fori_loop(0, V // 16, init_body, 0, unroll=8)
            cp.wait()

            ii = lax.iota(jnp.int32, 16)

            def body(t, _):
                vec = seq_v[pl.ds(t * 16, 16)]
                jg = base + t * 16 + ii
                c = (vec << 11) | jg
                s, _unused = plsc.sort_key_val(c, c)
                tmp_v[pl.ds(0, 16)] = s
                nxt = tmp_v[pl.ds(1, 16)]
                vs = s >> 11
                keep = vs != (nxt >> 11)
                plsc.store_scatter(lp_v, [vs], s & 2047, mask=keep)
                return 0

            lax.fori_loop(0, L // 32, body, 0, unroll=8)
            pltpu.sync_copy(lp_v, out_hbm.at[wid])

    return k(seq)


def _tc_body(lp_hbm, x_ref, o_ref, lp_vmem, sem):
    nl = pl.num_programs(1)
    bid = pl.program_id(0)
    pid = pl.program_id(1)

    @pl.when((bid == 0) & (pid == 0))
    def _():
        pltpu.make_async_copy(lp_hbm, lp_vmem, sem).start()
        pltpu.make_async_copy(lp_hbm, lp_vmem, sem).wait()

    x = x_ref[0]
    lps = jnp.maximum(lp_vmem[pl.ds(2 * bid, 1), :],
                      lp_vmem[pl.ds(2 * bid + 1, 1), :]) - pid * R
    ci = lax.broadcasted_iota(jnp.int32, (R, V), 0)
    m = jnp.max(x, axis=-1, keepdims=True)
    e = jnp.exp(x - m)

    last = jnp.logical_and(pid == nl - 1, ci == R - 1)
    cond = (lps >= ci) | last
    ez = jnp.where(cond, e, jnp.float32(0.0))
    o_ref[0] = ez / jnp.sum(ez, axis=-1, keepdims=True)


def _tc_softmax(inputs, lastpos):
    return pl.pallas_call(
        _tc_body,
        grid=(B, L // R),
        in_specs=[
            pl.BlockSpec(memory_space=pltpu.MemorySpace.HBM),
            pl.BlockSpec((1, R, V), lambda b, l: (b, l, 0)),
        ],
        out_specs=pl.BlockSpec((1, R, V), lambda b, l: (b, l, 0)),
        out_shape=jax.ShapeDtypeStruct((B, L, V), jnp.float32),
        scratch_shapes=[
            pltpu.VMEM((2 * B, V), jnp.int32),
            pltpu.SemaphoreType.DMA,
        ],
        compiler_params=pltpu.CompilerParams(
            dimension_semantics=("parallel", "parallel")),
    )(lastpos, inputs)


def kernel(inputs, seq):
    lastpos = _sc_lastpos(seq)
    return _tc_softmax(inputs, lastpos)


# speedup vs baseline: 1.0025x; 1.0025x over previous
import functools

import jax
import jax.numpy as jnp
import numpy as np
from jax import lax
from jax.experimental import pallas as pl
from jax.experimental.pallas import tpu as pltpu
from jax.experimental.pallas import tpu_sc as plsc

B, L, V = 8, 2048, 2048
NEG = float(np.log(np.float32(1e-45)))
R = 1024


def _sc_lastpos(seq):
    mesh = plsc.VectorSubcoreMesh(core_axis_name="c", subcore_axis_name="s", num_cores=1)

    @functools.partial(
        pl.kernel,
        mesh=mesh,
        out_type=jax.ShapeDtypeStruct((2 * B, V), jnp.int32),
        compiler_params=pltpu.CompilerParams(needs_layout_passes=False),
        scratch_types=[
            pltpu.VMEM((L // 2,), jnp.int32),
            pltpu.VMEM((V,), jnp.int32),
            pltpu.VMEM((32,), jnp.int32),
        ],
    )
    def k(seq_hbm, out_hbm, seq_v, lp_v, tmp_v):
        wid = lax.axis_index("s")

        @pl.when(wid < 2 * B)
        def _():
            b = wid // 2
            base = (wid % 2) * (L // 2)
            pltpu.sync_copy(seq_hbm.at[b, pl.ds(base, L // 2)], seq_v)
            tmp_v[pl.ds(16, 16)] = jnp.full((16,), 0x7FFFFFFF, jnp.int32)

            def init_body(i, _):
                lp_v[pl.ds(i * 16, 16)] = jnp.full((16,), -1, jnp.int32)
                return 0

            lax.fori_loop(0, V // 16, init_body, 0, unroll=8)

            ii = lax.iota(jnp.int32, 16)

            def body(t, _):
                vec = seq_v[pl.ds(t * 16, 16)]
                jg = base + t * 16 + ii
                c = (vec << 11) | jg
                s, _unused = plsc.sort_key_val(c, c)
                tmp_v[pl.ds(0, 16)] = s
                nxt = tmp_v[pl.ds(1, 16)]
                vs = s >> 11
                keep = vs != (nxt >> 11)
                plsc.store_scatter(lp_v, [vs], s & 2047, mask=keep)
                return 0

            lax.fori_loop(0, L // 32, body, 0, unroll=4)
            pltpu.sync_copy(lp_v, out_hbm.at[wid])

    return k(seq)


def _tc_body(lp_hbm, x_ref, o_ref, lp_vmem, sem):
    nl = pl.num_programs(1)
    bid = pl.program_id(0)
    pid = pl.program_id(1)

    @pl.when((bid == 0) & (pid == 0))
    def _():
        pltpu.make_async_copy(lp_hbm, lp_vmem, sem).start()
        pltpu.make_async_copy(lp_hbm, lp_vmem, sem).wait()

    x = x_ref[0]
    lps = jnp.maximum(lp_vmem[pl.ds(2 * bid, 1), :],
                      lp_vmem[pl.ds(2 * bid + 1, 1), :]) - pid * R
    ci = lax.broadcasted_iota(jnp.int32, (R, V), 0)
    m = jnp.max(x, axis=-1, keepdims=True)
    e = jnp.exp(x - m)

    last = jnp.logical_and(pid == nl - 1, ci == R - 1)
    cond = (lps >= ci) | last
    ez = jnp.where(cond, e, jnp.float32(0.0))
    o_ref[0] = ez / jnp.sum(ez, axis=-1, keepdims=True)


def _tc_softmax(inputs, lastpos):
    return pl.pallas_call(
        _tc_body,
        grid=(B, L // R),
        in_specs=[
            pl.BlockSpec(memory_space=pltpu.MemorySpace.HBM),
            pl.BlockSpec((1, R, V), lambda b, l: (b, l, 0)),
        ],
        out_specs=pl.BlockSpec((1, R, V), lambda b, l: (b, l, 0)),
        out_shape=jax.ShapeDtypeStruct((B, L, V), jnp.float32),
        scratch_shapes=[
            pltpu.VMEM((2 * B, V), jnp.int32),
            pltpu.SemaphoreType.DMA,
        ],
        compiler_params=pltpu.CompilerParams(
            dimension_semantics=("parallel", "parallel")),
    )(lastpos, inputs)


def kernel(inputs, seq):
    lastpos = _sc_lastpos(seq)
    return _tc_softmax(inputs, lastpos)
